# Initial kernel scaffold; baseline (speedup 1.0000x reference)
import jax
import jax.numpy as jnp
from jax.experimental import pallas as pl


def kernel(hop_distances, embedding_weight):
    raise NotImplementedError("write your pallas kernel here")


# speedup vs baseline: 1.1017x; 1.1017x over previous
import functools

import jax
import jax.numpy as jnp
from jax import lax
from jax.experimental import pallas as pl
from jax.experimental.pallas import tpu as pltpu
from jax.experimental.pallas import tpu_sc as plsc

NC = 2
NS = 16
NW = NC * NS
LANES = 16
CHUNK = 128


@functools.partial(jax.jit, static_argnames=("n_rows", "d"))
def _sc_lookup(idx_flat, table, *, n_rows, d):
    rows_per_w = n_rows // NW
    n_chunks = rows_per_w // CHUNK

    mesh = plsc.VectorSubcoreMesh(core_axis_name="c", subcore_axis_name="s")

    @functools.partial(
        pl.kernel,
        out_type=jax.ShapeDtypeStruct((n_rows, d), jnp.float32),
        mesh=mesh,
        scratch_types=[
            pltpu.VMEM((CHUNK,), jnp.int32),
            pltpu.VMEM((CHUNK,), jnp.int32),
            pltpu.VMEM((CHUNK, d), jnp.float32),
            pltpu.SemaphoreType.DMA,
        ],
        compiler_params=pltpu.CompilerParams(use_tc_tiling_on_sc=False),
    )
    def body(table_hbm, idx_hbm, out_hbm, idx_raw, idx_shift, rows, sem):
        wid = lax.axis_index("s") * NC + lax.axis_index("c")
        base = wid * rows_per_w

        def chunk_body(i, carry):
            off = base + i * CHUNK
            pltpu.sync_copy(idx_hbm.at[pl.ds(off, CHUNK)], idx_raw)
            for k in range(CHUNK // LANES):
                sl = pl.ds(k * LANES, LANES)
                idx_shift[sl] = idx_raw[sl] + 1
            pltpu.async_copy(table_hbm.at[idx_shift], rows, sem).wait()
            pltpu.sync_copy(rows, out_hbm.at[pl.ds(off, CHUNK)])
            return carry

        lax.fori_loop(0, n_chunks, chunk_body, 0)

    return body(table, idx_flat)


def kernel(hop_distances, embedding_weight):
    b, t = hop_distances.shape
    _, d = embedding_weight.shape
    idx_flat = hop_distances.astype(jnp.int32).reshape(-1)
    out = _sc_lookup(idx_flat, embedding_weight, n_rows=b * t, d=d)
    return out.reshape(b, t, d)
